# baseline (speedup 1.0000x reference)
import functools

import jax
import jax.numpy as jnp
from jax import lax
from jax.experimental import pallas as pl
from jax.experimental.pallas import tpu as pltpu
from jax.experimental.pallas import tpu_sc as plsc

SEP = 7
LANES = 16
ROWS_PER_CHUNK = 16


def _body(seq, batch, vocab, vpad, logits_hbm, di_hbm, w2s_hbm, out_hbm,
          di_v, w2s_v, rs_row_v, rs_bcast_v, in_bufs, out_bufs,
          sem_di, sem_tbl, sems_in, sems_out):
    my_s = lax.axis_index("c") * 16 + lax.axis_index("s")
    ngroups = batch // LANES
    nchunks = batch // ROWS_PER_CHUNK
    offs = [j * LANES for j in range(vocab // LANES)]
    if vocab % LANES:
        offs.append(vocab - LANES)

    cp_di = pltpu.make_async_copy(di_hbm, di_v, sem_di)
    cp_di.start()
    cp_tbl = pltpu.make_async_copy(w2s_hbm, w2s_v, sem_tbl)
    cp_tbl.start()

    def in_copy(k):
        return pltpu.make_async_copy(
            logits_hbm.at[my_s, pl.ds(k * ROWS_PER_CHUNK, ROWS_PER_CHUNK), :],
            in_bufs[k % 2],
            sems_in[k % 2])

    def out_copy(k):
        return pltpu.make_async_copy(
            out_bufs[k % 2],
            out_hbm.at[my_s, pl.ds(k * ROWS_PER_CHUNK, ROWS_PER_CHUNK), :],
            sems_out[k % 2])

    in_copy(0).start()
    in_copy(1).start()
    cp_di.wait()
    cp_tbl.wait()

    def full(v):
        return jnp.full((LANES,), v, jnp.int32)

    def group_body(w, carry):
        c0, c2, c5, c7 = full(0), full(2), full(5), full(7)
        rs = c0
        seg = c0
        snap = c0
        for t in range(seq):
            tok = di_v[t, pl.ds(w * LANES, LANES)]
            is_sep = tok == c7
            sep_i = jnp.where(is_sep, full(1), c0)
            if t == 0:
                seg = sep_i
                rs = jnp.where(is_sep, c7, c5)
            else:
                syl = plsc.load_gather(w2s_v, [tok])
                rs = jnp.maximum(rs - syl, c0)
                seg = jnp.minimum(seg + sep_i, c5)
                pat = jnp.where(seg == c5, c0,
                                jnp.where((seg == c0) | (seg == c2), c5, c7))
                rs = jnp.where(is_sep, pat, rs)
            snap = jnp.where(t == my_s, rs, snap)
        rs_row_v[pl.ds(w * LANES, LANES)] = snap
        return carry

    lax.fori_loop(0, ngroups, group_body, 0)

    def splat_body(b, carry):
        idx = jnp.broadcast_to(b, (LANES,)).astype(jnp.int32)
        rs_bcast_v[b] = plsc.load_gather(rs_row_v, [idx])
        return carry

    lax.fori_loop(0, batch, splat_body, 0)

    minf = jnp.full((LANES,), -jnp.inf, jnp.float32)

    for k in range(nchunks):
        in_copy(k).wait()
        if k >= 2:
            out_copy(k - 2).wait()
        inb = in_bufs[k % 2]
        outb = out_bufs[k % 2]
        base = k * ROWS_PER_CHUNK

        def row_body(b, carry, inb=inb, outb=outb, base=base):
            rsv = rs_bcast_v[base + b]
            for off in offs:
                w2s_c = w2s_v[pl.ds(off, LANES)]
                x = inb[b, pl.ds(off, LANES)]
                outb[b, pl.ds(off, LANES)] = jnp.where(
                    w2s_c > rsv, minf, x)
            return carry

        lax.fori_loop(0, ROWS_PER_CHUNK, row_body, 0)
        out_copy(k).start()
        if k + 2 < nchunks:
            in_copy(k + 2).start()

    out_copy(nchunks - 2).wait()
    out_copy(nchunks - 1).wait()


def kernel(logits, decoder_input, word2syllables):
    seq, batch = decoder_input.shape
    vocab = logits.shape[-1]
    vpad = (-vocab) % LANES
    w2s_pad = jnp.concatenate(
        [word2syllables, jnp.zeros((vpad,), word2syllables.dtype)])

    fn = pl.kernel(
        functools.partial(_body, seq, batch, vocab, vpad),
        out_type=jax.ShapeDtypeStruct((seq, batch, vocab), jnp.float32),
        mesh=plsc.VectorSubcoreMesh(core_axis_name="c", subcore_axis_name="s"),
        compiler_params=pltpu.CompilerParams(needs_layout_passes=False),
        scratch_types=[
            pltpu.VMEM((seq, batch), jnp.int32),
            pltpu.VMEM((vocab + vpad,), jnp.int32),
            pltpu.VMEM((batch,), jnp.int32),
            pltpu.VMEM((batch, LANES), jnp.int32),
            [pltpu.VMEM((ROWS_PER_CHUNK, vocab), jnp.float32)
             for _ in range(2)],
            [pltpu.VMEM((ROWS_PER_CHUNK, vocab), jnp.float32)
             for _ in range(2)],
            pltpu.SemaphoreType.DMA,
            pltpu.SemaphoreType.DMA,
            [pltpu.SemaphoreType.DMA for _ in range(2)],
            [pltpu.SemaphoreType.DMA for _ in range(2)],
        ],
    )
    return fn(logits, decoder_input, w2s_pad)


# speedup vs baseline: 1.5032x; 1.5032x over previous
import jax
import jax.numpy as jnp
from jax.experimental import pallas as pl


def _body(di_ref, w2s_ref, w2spad_ref, logits_ref, out_ref, rs_ref, seg_ref):
    s = pl.program_id(0)
    nsteps = pl.num_programs(0)
    del nsteps

    colmask = jax.lax.broadcasted_iota(jnp.int32, di_ref.shape, 1) == s
    tok = jnp.sum(jnp.where(colmask, di_ref[...], 0), axis=1, keepdims=True)

    w2s_row = w2spad_ref[...]
    volane = jax.lax.broadcasted_iota(jnp.int32, w2s_row.shape, 1)
    onehot = volane == tok
    syl = jnp.sum(jnp.where(onehot, w2s_row, 0), axis=1, keepdims=True)

    is_sep = tok == 7
    sep_i = is_sep.astype(jnp.int32)
    first = s == 0

    seg = jnp.where(first, sep_i, jnp.minimum(seg_ref[...] + sep_i, 5))
    rs_dec = jnp.maximum(rs_ref[...] - syl, 0)
    pat = jnp.where(seg == 5, 0, jnp.where((seg == 0) | (seg == 2), 5, 7))
    sep_val = jnp.where(first, 7, pat)
    rs = jnp.where(is_sep, sep_val, jnp.where(first, 5, rs_dec))
    rs_ref[...] = rs
    seg_ref[...] = seg

    out_ref[0] = jnp.where(w2s_ref[...] > rs, -jnp.inf, logits_ref[0])


def kernel(logits, decoder_input, word2syllables):
    seq, batch = decoder_input.shape
    vocab = logits.shape[-1]
    from jax.experimental.pallas import tpu as pltpu

    di_t = decoder_input.T
    w2s2 = word2syllables.reshape(1, vocab)
    vpad = (-vocab) % 128
    w2s_padded = jnp.concatenate(
        [word2syllables, jnp.full((vpad,), -1, word2syllables.dtype)]
    ).reshape(1, vocab + vpad)

    out = pl.pallas_call(
        _body,
        grid=(seq,),
        in_specs=[
            pl.BlockSpec((batch, seq), lambda i: (0, 0)),
            pl.BlockSpec((1, vocab), lambda i: (0, 0)),
            pl.BlockSpec((1, vocab + vpad), lambda i: (0, 0)),
            pl.BlockSpec((1, batch, vocab), lambda i: (i, 0, 0)),
        ],
        out_specs=pl.BlockSpec((1, batch, vocab), lambda i: (i, 0, 0)),
        out_shape=jax.ShapeDtypeStruct((seq, batch, vocab), jnp.float32),
        scratch_shapes=[
            pltpu.VMEM((batch, 1), jnp.int32),
            pltpu.VMEM((batch, 1), jnp.int32),
        ],
    )(di_t, w2s2, w2s_padded, logits)
    return out


# speedup vs baseline: 1.9144x; 1.2736x over previous
import functools

import jax
import jax.numpy as jnp
from jax.experimental import pallas as pl
from jax.experimental.pallas import tpu as pltpu

ROWS = 8


def _body(rows, di_ref, w2s_ref, pk_ref, logits_ref, out_ref, rs_ref, seg_ref):
    i = pl.program_id(0)
    first_block = i == 0

    packed_row = pk_ref[...]
    lanes = jax.lax.broadcasted_iota(jnp.int32, packed_row.shape, 1)
    w2s_row = w2s_ref[...]

    rs = rs_ref[...]
    seg = seg_ref[...]

    for r in range(rows):
        tok = di_ref[r]
        is_sep = tok == 7
        sep_i = is_sep.astype(jnp.int32)

        widx = jax.lax.shift_right_logical(tok, 3)
        psel = jnp.sum(jnp.where(lanes == widx, packed_row, 0),
                       axis=1, keepdims=True)
        syl = jax.lax.shift_right_logical(psel, (tok & 7) * 4) & 15

        if r == 0:
            seg = jnp.where(first_block, sep_i,
                            jnp.minimum(seg + sep_i, 5))
        else:
            seg = jnp.minimum(seg + sep_i, 5)
        rs_dec = jnp.maximum(rs - syl, 0)
        pat = jnp.where(seg == 5, 0, jnp.where((seg == 0) | (seg == 2), 5, 7))
        if r == 0:
            sep_val = jnp.where(first_block, 7, pat)
            rs = jnp.where(is_sep, sep_val,
                           jnp.where(first_block, 5, rs_dec))
        else:
            rs = jnp.where(is_sep, pat, rs_dec)

        out_ref[r] = jnp.where(w2s_row > rs, -jnp.inf, logits_ref[r])

    rs_ref[...] = rs
    seg_ref[...] = seg


def kernel(logits, decoder_input, word2syllables):
    seq, batch = decoder_input.shape
    vocab = logits.shape[-1]

    di3 = decoder_input.reshape(seq, batch, 1)
    w2s2 = word2syllables.reshape(1, vocab)
    npk = (vocab + 7) // 8
    w2s_grp = jnp.zeros((npk * 8,), jnp.int32).at[:vocab].set(
        word2syllables.astype(jnp.int32) & 15).reshape(npk, 8)
    shifts = (jnp.arange(8, dtype=jnp.int32) * 4)[None, :]
    packed = jnp.sum(w2s_grp << shifts, axis=1).astype(jnp.int32)
    npk_pad = (-npk) % 128
    packed_padded = jnp.concatenate(
        [packed, jnp.zeros((npk_pad,), jnp.int32)]).reshape(1, npk + npk_pad)

    out = pl.pallas_call(
        functools.partial(_body, ROWS),
        grid=(seq // ROWS,),
        in_specs=[
            pl.BlockSpec((ROWS, batch, 1), lambda i: (i, 0, 0)),
            pl.BlockSpec((1, vocab), lambda i: (0, 0)),
            pl.BlockSpec((1, npk + npk_pad), lambda i: (0, 0)),
            pl.BlockSpec((ROWS, batch, vocab), lambda i: (i, 0, 0)),
        ],
        out_specs=pl.BlockSpec((ROWS, batch, vocab), lambda i: (i, 0, 0)),
        out_shape=jax.ShapeDtypeStruct((seq, batch, vocab), jnp.float32),
        scratch_shapes=[
            pltpu.VMEM((batch, 1), jnp.int32),
            pltpu.VMEM((batch, 1), jnp.int32),
        ],
    )(di3, w2s2, packed_padded, logits)
    return out
